# baseline (speedup 1.0000x reference)
import functools

import jax
import jax.numpy as jnp
from jax import lax
from jax.experimental import pallas as pl
from jax.experimental.pallas import tpu as pltpu
from jax.experimental.pallas import tpu_sc as plsc

NUM_BINS = 61928
EMBED_DIM = 16
BATCH = 16384
SPAN = 8

_info = plsc.get_sparse_core_info()
NC, NS, L = _info.num_cores, _info.num_subcores, _info.num_lanes
NPAIR = EMBED_DIM // 2
NHALF = NS // NPAIR
NQUARTER = NC * NHALF
FEAT_PER_Q = BATCH // NQUARTER
FCHUNK = 1024
NFCHUNK = FEAT_PER_Q // FCHUNK
GROUPS = FCHUNK // L


def _make_kernel():
    mesh = plsc.VectorSubcoreMesh(core_axis_name="c", subcore_axis_name="s")

    @functools.partial(
        pl.kernel,
        mesh=mesh,
        out_type=jax.ShapeDtypeStruct((NPAIR, 2, BATCH), jnp.float32),
        scratch_types=[
            pltpu.VMEM((NUM_BINS,), jnp.int32),
            pltpu.VMEM((2, SPAN, FCHUNK), jnp.int32),
            pltpu.VMEM((2, FEAT_PER_Q), jnp.float32),
            pltpu.SemaphoreType.DMA,
            pltpu.SemaphoreType.DMA,
            pltpu.SemaphoreType.DMA,
        ],
        compiler_params=pltpu.CompilerParams(
            use_tc_tiling_on_sc=False, needs_layout_passes=False
        ),
    )
    def k(idx_hbm, tab_hbm, out_hbm, colp_v, idx_v, out_v, sem_a, sem_b, sem_c):
        sid = lax.axis_index("s")
        pair = sid % NPAIR
        half = sid // NPAIR
        rep = lax.axis_index("c")
        feat_base = (rep * NHALF + half) * FEAT_PER_Q
        sems = (sem_a, sem_b)

        def idx_copy(chunk, buf):
            return pltpu.make_async_copy(
                idx_hbm.at[:, pl.ds(feat_base + chunk * FCHUNK, FCHUNK)],
                idx_v.at[buf],
                sems[buf],
            )

        col_copy = pltpu.make_async_copy(tab_hbm.at[pair], colp_v, sem_c)
        col_copy.start()
        idx_copy(0, 0).start()
        col_copy.wait()

        inv = jnp.float32(1.0 / SPAN)

        def run_groups(buf, chunk):
            out_base = chunk * FCHUNK

            @plsc.parallel_loop(0, GROUPS, unroll=2)
            def _groups(g):
                f0 = g * L
                acc0 = jnp.zeros((L,), jnp.float32)
                acc1 = jnp.zeros((L,), jnp.float32)
                for s in range(SPAN):
                    bins = idx_v[buf, s, pl.ds(f0, L)]
                    w = plsc.load_gather(colp_v, [bins])
                    bf = plsc.bitcast(w, jnp.bfloat16)
                    a, b = plsc.unpack(bf, format=plsc.PackFormat.INTERLEAVED)
                    acc0 = acc0 + a
                    acc1 = acc1 + b
                out_v[0, pl.ds(out_base + f0, L)] = acc0 * inv
                out_v[1, pl.ds(out_base + f0, L)] = acc1 * inv

        def pair_body(c, carry):
            chunk0 = 2 * c
            chunk1 = 2 * c + 1

            idx_copy(chunk1, 1).start()
            idx_copy(chunk0, 0).wait()
            run_groups(0, chunk0)

            @pl.when(c + 1 < NFCHUNK // 2)
            def _():
                idx_copy(chunk1 + 1, 0).start()

            idx_copy(chunk1, 1).wait()
            run_groups(1, chunk1)
            return carry

        lax.fori_loop(0, NFCHUNK // 2, pair_body, 0, unroll=False)

        pltpu.sync_copy(
            out_v, out_hbm.at[pair, :, pl.ds(feat_base, FEAT_PER_Q)]
        )

    return k


_sc_kernel = _make_kernel()


def kernel(bin_idxs, table):
    idx_t = jnp.transpose(bin_idxs.astype(jnp.int32))
    tab_bf = table.astype(jnp.bfloat16)
    packed = lax.bitcast_convert_type(
        tab_bf.reshape(NUM_BINS, NPAIR, 2), jnp.int32
    )
    packed_t = jnp.transpose(packed)
    parts = _sc_kernel(idx_t, packed_t)
    return jnp.transpose(parts, (2, 0, 1)).reshape(BATCH, EMBED_DIM)


# speedup vs baseline: 1.1231x; 1.1231x over previous
import functools

import jax
import jax.numpy as jnp
from jax import lax
from jax.experimental import pallas as pl
from jax.experimental.pallas import tpu as pltpu
from jax.experimental.pallas import tpu_sc as plsc

NUM_BINS = 61928
EMBED_DIM = 16
BATCH = 16384
SPAN = 8

_info = plsc.get_sparse_core_info()
NC, NS, L = _info.num_cores, _info.num_subcores, _info.num_lanes
NREP = NC
FEAT_PER_REP = BATCH // NREP
FCHUNK = 1024
NFCHUNK = FEAT_PER_REP // FCHUNK
GROUPS = FCHUNK // L


def _make_kernel():
    mesh = plsc.VectorSubcoreMesh(core_axis_name="c", subcore_axis_name="s")

    @functools.partial(
        pl.kernel,
        mesh=mesh,
        out_type=jax.ShapeDtypeStruct((EMBED_DIM, BATCH), jnp.float32),
        scratch_types=[
            pltpu.VMEM((NUM_BINS,), jnp.float32),
            pltpu.VMEM((2, SPAN, FCHUNK), jnp.int32),
            pltpu.VMEM((FEAT_PER_REP,), jnp.float32),
            pltpu.SemaphoreType.DMA,
            pltpu.SemaphoreType.DMA,
            pltpu.SemaphoreType.DMA,
        ],
        compiler_params=pltpu.CompilerParams(
            use_tc_tiling_on_sc=False, needs_layout_passes=False
        ),
    )
    def k(idx_hbm, tab_hbm, out_hbm, col_v, idx_v, out_v, sem_a, sem_b, sem_c):
        col_id = lax.axis_index("s")
        rep = lax.axis_index("c")
        feat_base = rep * FEAT_PER_REP
        sems = (sem_a, sem_b)

        def idx_copy(chunk, buf):
            return pltpu.make_async_copy(
                idx_hbm.at[:, pl.ds(feat_base + chunk * FCHUNK, FCHUNK)],
                idx_v.at[buf],
                sems[buf],
            )

        col_copy = pltpu.make_async_copy(tab_hbm.at[col_id], col_v, sem_c)
        col_copy.start()
        idx_copy(0, 0).start()
        col_copy.wait()

        inv = jnp.float32(1.0 / SPAN)

        def run_groups(buf, chunk):
            out_base = chunk * FCHUNK

            @plsc.parallel_loop(0, GROUPS, unroll=4)
            def _groups(g):
                f0 = g * L
                bins = idx_v[buf, 0, pl.ds(f0, L)]
                acc = plsc.load_gather(col_v, [bins])
                for s in range(1, SPAN):
                    bins = idx_v[buf, s, pl.ds(f0, L)]
                    acc = acc + plsc.load_gather(col_v, [bins])
                out_v[pl.ds(out_base + f0, L)] = acc * inv

        def pair_body(c, carry):
            chunk0 = 2 * c
            chunk1 = 2 * c + 1

            idx_copy(chunk1, 1).start()
            idx_copy(chunk0, 0).wait()
            run_groups(0, chunk0)

            @pl.when(c + 1 < NFCHUNK // 2)
            def _():
                idx_copy(chunk1 + 1, 0).start()

            idx_copy(chunk1, 1).wait()
            run_groups(1, chunk1)
            return carry

        lax.fori_loop(0, NFCHUNK // 2, pair_body, 0, unroll=False)

        pltpu.sync_copy(out_v, out_hbm.at[col_id, pl.ds(feat_base, FEAT_PER_REP)])

    return k


_sc_kernel = _make_kernel()


def kernel(bin_idxs, table):
    idx_t = jnp.transpose(bin_idxs.astype(jnp.int32))
    tab_t = jnp.transpose(table)
    parts = _sc_kernel(idx_t, tab_t)
    return jnp.transpose(parts)
